# baseline (speedup 1.0000x reference)
import jax
import jax.numpy as jnp
from jax import lax
from jax.experimental import pallas as pl
from jax.experimental.pallas import tpu as pltpu
from jax.experimental.pallas import tpu_sc as plsc

B, T, N, D = 8, 4, 1024, 64
H, W, P = 512, 512, 4096
BT = B * T
L = 16


def _rsqrt(x):
    i = lax.bitcast_convert_type(x, jnp.int32)
    i = 0x5F3759DF - lax.shift_right_arithmetic(i, 1)
    y = lax.bitcast_convert_type(i, jnp.float32)
    for _ in range(3):
        y = y * (1.5 - 0.5 * x * y * y)
    return y


def _body(lat_h, hws_h, val_h, seg_h, si_h,
          dep_h, img_h, nrm_h, vv_h,
          lat_v, hws_v, val_v, si_v, hf_v, wf_v, idx_v, seg_v,
          dep_v, img_v, nrm_v, vv_v,
          sem_tab, sem_pix, sem_seg):
    cidx = lax.axis_index("c")
    sidx = lax.axis_index("s")
    bt = sidx * 2 + cidx

    cp_lat = pltpu.async_copy(lat_h.at[bt], lat_v, sem_tab)
    cp_hws = pltpu.async_copy(hws_h.at[bt], hws_v, sem_tab)
    cp_val = pltpu.async_copy(val_h.at[bt], val_v, sem_tab)
    cp_si = pltpu.async_copy(si_h.at[bt], si_v, sem_pix)
    cp_si.wait()

    base_img = bt * (H * W)
    iota = lax.broadcasted_iota(jnp.int32, (L,), 0)

    @plsc.parallel_loop(0, P, step=L, unroll=4)
    def _mk_idx(o):
        pix2 = (iota + o) * 2
        h = plsc.load_gather(si_v, [pix2])
        w = plsc.load_gather(si_v, [pix2 + 1])
        idx_v[pl.ds(o, L)] = h * W + w + base_img
        hf_v[pl.ds(o, L)] = h.astype(jnp.float32) * (2.0 / (H - 1)) - 1.0
        wf_v[pl.ds(o, L)] = w.astype(jnp.float32) * (2.0 / (W - 1)) - 1.0

    cp_seg = pltpu.async_copy(seg_h.at[idx_v], seg_v, sem_seg)
    cp_lat.wait()
    cp_hws.wait()
    cp_val.wait()
    cp_seg.wait()

    @plsc.parallel_loop(0, P, step=L, unroll=2)
    def _decode(o):
        segj = seg_v[pl.ds(o, L)]
        vmask = (segj >= 0) & (segj < N)
        sg = lax.min(lax.max(segj, 0), N - 1)
        hf = hf_v[pl.ds(o, L)]
        wf = wf_v[pl.ds(o, L)]
        cen_h = plsc.load_gather(hws_v, [sg * 2])
        cen_w = plsc.load_gather(hws_v, [sg * 2 + 1])
        vn = plsc.load_gather(val_v, [sg])
        vv = jnp.where(vmask, vn, 0.0)
        dH = hf - cen_h
        dW = wf - cen_w
        d3 = dH * dH
        d4 = dH * dW
        d5 = dW * dW
        s64 = sg * D

        def acc(ch0, stride):
            a = [plsc.load_gather(lat_v, [s64 + (ch0 + i * stride)])
                 for i in range(6)]
            return ((a[0] + a[1] * dH) + (a[2] * dW + a[3] * d3)
                    + (a[4] * d4 + a[5] * d5))

        dep = jnp.minimum(acc(0, 1) * vv, -0.1)
        dep_v[pl.ds(o, L)] = dep
        vv_v[pl.ds(o, L)] = vv

        sidx3 = iota * 3 + (o * 3)
        for ci in range(3):
            u = acc(6 + ci, 3) * vv
            plsc.store_scatter(img_v, [sidx3 + ci], jnp.clip(u, -100.0, 100.0))
        w0 = acc(24, 3) * vv
        w1 = acc(25, 3) * vv
        w2 = acc(26, 3) * vv
        r = _rsqrt(jnp.maximum(w0 * w0 + w1 * w1 + w2 * w2, 1e-12))
        plsc.store_scatter(nrm_v, [sidx3], w0 * r)
        plsc.store_scatter(nrm_v, [sidx3 + 1], w1 * r)
        plsc.store_scatter(nrm_v, [sidx3 + 2], w2 * r)

    pltpu.sync_copy(dep_v, dep_h.at[bt])
    pltpu.sync_copy(img_v, img_h.at[bt])
    pltpu.sync_copy(nrm_v, nrm_h.at[bt])
    pltpu.sync_copy(vv_v, vv_h.at[bt])


def kernel(latent_vec, node_hws, valid_nodes, segment_ids, spatial_inds):
    lat = latent_vec.reshape(BT, N * D)
    hws = node_hws.reshape(BT, N * 2)
    val = valid_nodes.reshape(BT, N)
    seg = segment_ids.reshape(BT * H * W)
    si = spatial_inds.reshape(BT, 2 * P)

    mesh = plsc.VectorSubcoreMesh(core_axis_name="c", subcore_axis_name="s",
                                  num_cores=2, num_subcores=16)
    f = pl.kernel(
        _body,
        out_type=(
            jax.ShapeDtypeStruct((BT, P), jnp.float32),
            jax.ShapeDtypeStruct((BT, 3 * P), jnp.float32),
            jax.ShapeDtypeStruct((BT, 3 * P), jnp.float32),
            jax.ShapeDtypeStruct((BT, P), jnp.float32),
        ),
        mesh=mesh,
        compiler_params=pltpu.CompilerParams(needs_layout_passes=False),
        scratch_types=[
            pltpu.VMEM((N * D,), jnp.float32),
            pltpu.VMEM((2 * N,), jnp.float32),
            pltpu.VMEM((N,), jnp.float32),
            pltpu.VMEM((2 * P,), jnp.int32),
            pltpu.VMEM((P,), jnp.float32),
            pltpu.VMEM((P,), jnp.float32),
            pltpu.VMEM((P,), jnp.int32),
            pltpu.VMEM((P,), jnp.int32),
            pltpu.VMEM((P,), jnp.float32),
            pltpu.VMEM((3 * P,), jnp.float32),
            pltpu.VMEM((3 * P,), jnp.float32),
            pltpu.VMEM((P,), jnp.float32),
            pltpu.SemaphoreType.DMA,
            pltpu.SemaphoreType.DMA,
            pltpu.SemaphoreType.DMA,
        ],
    )
    dep, img, nrm, vv = f(lat, hws, val, seg, si)
    return (dep.reshape(B, T, P, 1),
            img.reshape(B, T, P, 3),
            nrm.reshape(B, T, P, 3),
            vv.reshape(B, T, P, 1))


# speedup vs baseline: 6.0359x; 6.0359x over previous
import jax
import jax.numpy as jnp
from jax import lax
from jax.experimental import pallas as pl
from jax.experimental.pallas import tpu as pltpu
from jax.experimental.pallas import tpu_sc as plsc

B, T, N, D = 8, 4, 1024, 64
H, W, P = 512, 512, 4096
BT = B * T
L = 16


def _rsqrt(x):
    i = lax.bitcast_convert_type(x, jnp.int32)
    i = 0x5F3759DF - lax.shift_right_arithmetic(i, 1)
    y = lax.bitcast_convert_type(i, jnp.float32)
    for _ in range(3):
        y = y * (1.5 - 0.5 * x * y * y)
    return y


def _lat_off(c):
    return (c >> 3) * 8192 + (c & 7) * 128


def _body(lat_h, hws_h, val_h, seg_h, si_h,
          dep_h, img_h, nrm_h, vv_h,
          lat_v, hws_v, val_v, si_v, idx_v, seg_v,
          dep_v, vv_v, i0_v, i1_v, i2_v, n0_v, n1_v, n2_v,
          sem_tab, sem_pix, sem_seg):
    cidx = lax.axis_index("c")
    sidx = lax.axis_index("s")
    bt = sidx * 2 + cidx
    bi = bt >> 2
    ti = bt & 3

    cp_lat = pltpu.async_copy(lat_h.at[pl.ds(bt * (N * D), N * D)], lat_v, sem_tab)
    cp_hws = pltpu.async_copy(hws_h.at[pl.ds(bt * (2 * N), 2 * N)], hws_v, sem_tab)
    cp_val = pltpu.async_copy(val_h.at[pl.ds(bt * N, N)], val_v, sem_tab)
    cp_si = pltpu.async_copy(si_h.at[pl.ds(bt * (2 * P), 2 * P)], si_v, sem_pix)
    cp_si.wait()

    base_img = bt * (H * W)

    @plsc.parallel_loop(0, P, step=L, unroll=4)
    def _mk_idx(o):
        so = ((o >> 7) << 8) + (o & 127)
        hi = si_v[pl.ds(so, L)]
        wi = si_v[pl.ds(so + 128, L)]
        idx_v[pl.ds(o, L)] = (base_img + ((hi >> 3) << 12) + ((wi >> 7) << 10)
                              + ((hi & 7) << 7) + (wi & 127))

    cp_seg = pltpu.async_copy(seg_h.at[idx_v], seg_v, sem_seg)
    cp_lat.wait()
    cp_hws.wait()
    cp_val.wait()
    cp_seg.wait()

    @plsc.parallel_loop(0, P, step=L, unroll=2)
    def _decode(o):
        segj = seg_v[pl.ds(o, L)]
        vmask = (segj >= 0) & (segj < N)
        sg = lax.min(lax.max(segj, 0), N - 1)
        so = ((o >> 7) << 8) + (o & 127)
        hf = si_v[pl.ds(so, L)].astype(jnp.float32) * (2.0 / (H - 1)) - 1.0
        wf = si_v[pl.ds(so + 128, L)].astype(jnp.float32) * (2.0 / (W - 1)) - 1.0
        n_hi = sg >> 7
        n_lo = sg & 127
        tb_hws = (n_hi << 8) + n_lo
        tb_lat = (n_hi << 10) + n_lo
        cen_h = plsc.load_gather(hws_v, [tb_hws])
        cen_w = plsc.load_gather(hws_v, [tb_hws + 128])
        vn = plsc.load_gather(val_v, [sg])
        vv = jnp.where(vmask, vn, 0.0)
        dH = hf - cen_h
        dW = wf - cen_w
        d3 = dH * dH
        d4 = dH * dW
        d5 = dW * dW

        def acc(ch0, stride):
            a = [plsc.load_gather(lat_v, [tb_lat + _lat_off(ch0 + i * stride)])
                 for i in range(6)]
            return ((a[0] + a[1] * dH) + (a[2] * dW + a[3] * d3)
                    + (a[4] * d4 + a[5] * d5))

        dep_v[pl.ds(o, L)] = jnp.minimum(acc(0, 1) * vv, -0.1)
        vv_v[pl.ds(o, L)] = vv

        for ref, ci in ((i0_v, 0), (i1_v, 1), (i2_v, 2)):
            u = acc(6 + ci, 3) * vv
            ref[pl.ds(o, L)] = jnp.clip(u, -100.0, 100.0)
        w0 = acc(24, 3) * vv
        w1 = acc(25, 3) * vv
        w2 = acc(26, 3) * vv
        r = _rsqrt(jnp.maximum(w0 * w0 + w1 * w1 + w2 * w2, 1e-12))
        n0_v[pl.ds(o, L)] = w0 * r
        n1_v[pl.ds(o, L)] = w1 * r
        n2_v[pl.ds(o, L)] = w2 * r

    pltpu.sync_copy(dep_v, dep_h.at[pl.ds(bt * P, P)])
    pltpu.sync_copy(vv_v, vv_h.at[pl.ds(bt * P, P)])
    for c, ref in enumerate((i0_v, i1_v, i2_v)):
        pltpu.sync_copy(ref, img_h.at[pl.ds(((bi * 3 + c) * T + ti) * P, P)])
    for c, ref in enumerate((n0_v, n1_v, n2_v)):
        pltpu.sync_copy(ref, nrm_h.at[pl.ds(((bi * 3 + c) * T + ti) * P, P)])


def kernel(latent_vec, node_hws, valid_nodes, segment_ids, spatial_inds):
    lat = latent_vec.reshape(B, T, 8, 128, 8, 8).transpose(0, 1, 4, 2, 5, 3).reshape(-1)
    hws = node_hws.reshape(B, T, 8, 128, 2).transpose(0, 1, 2, 4, 3).reshape(-1)
    val = valid_nodes.reshape(-1)
    seg = segment_ids.reshape(B, T, 64, 8, 4, 128).transpose(0, 1, 2, 4, 3, 5).reshape(-1)
    si = spatial_inds.reshape(B, T, 32, 128, 2).transpose(0, 1, 2, 4, 3).reshape(-1)

    mesh = plsc.VectorSubcoreMesh(core_axis_name="c", subcore_axis_name="s",
                                  num_cores=2, num_subcores=16)
    f = pl.kernel(
        _body,
        out_type=(
            jax.ShapeDtypeStruct((BT * P,), jnp.float32),
            jax.ShapeDtypeStruct((B * 3 * T * P,), jnp.float32),
            jax.ShapeDtypeStruct((B * 3 * T * P,), jnp.float32),
            jax.ShapeDtypeStruct((BT * P,), jnp.float32),
        ),
        mesh=mesh,
        compiler_params=pltpu.CompilerParams(needs_layout_passes=False),
        scratch_types=[
            pltpu.VMEM((N * D,), jnp.float32),
            pltpu.VMEM((2 * N,), jnp.float32),
            pltpu.VMEM((N,), jnp.float32),
            pltpu.VMEM((2 * P,), jnp.int32),
            pltpu.VMEM((P,), jnp.int32),
            pltpu.VMEM((P,), jnp.int32),
            pltpu.VMEM((P,), jnp.float32),
            pltpu.VMEM((P,), jnp.float32),
            pltpu.VMEM((P,), jnp.float32),
            pltpu.VMEM((P,), jnp.float32),
            pltpu.VMEM((P,), jnp.float32),
            pltpu.VMEM((P,), jnp.float32),
            pltpu.VMEM((P,), jnp.float32),
            pltpu.VMEM((P,), jnp.float32),
            pltpu.SemaphoreType.DMA,
            pltpu.SemaphoreType.DMA,
            pltpu.SemaphoreType.DMA,
        ],
    )
    dep, img, nrm, vv = f(lat, hws, val, seg, si)
    return (dep.reshape(B, T, P, 1),
            img.reshape(B, 3, T, P).transpose(0, 2, 3, 1),
            nrm.reshape(B, 3, T, P).transpose(0, 2, 3, 1),
            vv.reshape(B, T, P, 1))
